# baseline (speedup 1.0000x reference)
import functools

import jax
import jax.numpy as jnp
from jax import lax
from jax.experimental import pallas as pl
from jax.experimental.pallas import tpu as pltpu
from jax.experimental.pallas import tpu_sc as plsc

VOCAB = 1000000
EMBED_DIM = 32
BATCH = 16384
HIST = 50
TOTAL = BATCH * HIST

_NUM_WORKERS = 32
_STRIPE = BATCH // _NUM_WORKERS

_mesh = plsc.VectorSubcoreMesh(core_axis_name="c", subcore_axis_name="s")


@functools.partial(
    pl.kernel,
    mesh=_mesh,
    out_type=jax.ShapeDtypeStruct((HIST, EMBED_DIM, BATCH), jnp.float32),
    scratch_types=[
        pltpu.VMEM((2, _STRIPE), jnp.int32),
        pltpu.VMEM((2, _STRIPE, EMBED_DIM), jnp.float32),
        pltpu.VMEM((2, EMBED_DIM, _STRIPE), jnp.float32),
        pltpu.SemaphoreType.DMA((2,)),
        pltpu.SemaphoreType.DMA((2,)),
    ],
    compiler_params=pltpu.CompilerParams(use_tc_tiling_on_sc=False,
                                         needs_layout_passes=False),
)
def _gather_kernel(idx_hbm, table_hbm, out_hbm, idx_v, rows_v, trows_v,
                   gsems, osems):
    wid = lax.axis_index("s") * 2 + lax.axis_index("c")
    col0 = wid * _STRIPE
    iota16 = lax.iota(jnp.int32, 16)

    def start_gather(h, b):
        pltpu.sync_copy(idx_hbm.at[pl.ds(h * BATCH + col0, _STRIPE)],
                        idx_v.at[b])
        pltpu.make_async_copy(table_hbm.at[idx_v.at[b]], rows_v.at[b],
                              gsems.at[b]).start()

    def out_copy(h, b):
        return pltpu.make_async_copy(
            trows_v.at[b],
            out_hbm.at[h, :, pl.ds(col0, _STRIPE)],
            osems.at[b])

    def transpose_block(b):
        def jbody(j, carry):
            for jj in range(8):
                rid = iota16 + (j * 8 + jj) * 16
                for e in range(EMBED_DIM):
                    col = jnp.full((16,), e, jnp.int32)
                    v = plsc.load_gather(rows_v.at[b], [rid, col])
                    trows_v.at[b][e, pl.ds((j * 8 + jj) * 16, 16)] = v
            return carry
        lax.fori_loop(0, _STRIPE // 128, jbody, 0)

    start_gather(0, 0)

    def slot(h, b):
        @pl.when(h + 1 < HIST)
        def _():
            start_gather(h + 1, 1 - b)
        pltpu.make_async_copy(table_hbm.at[idx_v.at[b]], rows_v.at[b],
                              gsems.at[b]).wait()

        @pl.when(h >= 2)
        def _():
            out_copy(h - 2, b).wait()
        transpose_block(b)
        out_copy(h, b).start()

    def gbody(g, carry):
        slot(2 * g, 0)
        slot(2 * g + 1, 1)
        return carry

    lax.fori_loop(0, HIST // 2, gbody, 0)
    out_copy(HIST - 2, 0).wait()
    out_copy(HIST - 1, 1).wait()


def kernel(x, vectors):
    idx = x.T.reshape(-1).astype(jnp.int32)
    out = _gather_kernel(idx, vectors)
    return out.transpose(2, 0, 1)


# speedup vs baseline: 1.1892x; 1.1892x over previous
import functools

import jax
import jax.numpy as jnp
from jax import lax
from jax.experimental import pallas as pl
from jax.experimental.pallas import tpu as pltpu
from jax.experimental.pallas import tpu_sc as plsc

VOCAB = 1000000
EMBED_DIM = 32
BATCH = 16384
HIST = 50
TOTAL = BATCH * HIST

_NUM_WORKERS = 32
_STRIPE = BATCH // _NUM_WORKERS

_mesh = plsc.VectorSubcoreMesh(core_axis_name="c", subcore_axis_name="s")


@functools.partial(
    pl.kernel,
    mesh=_mesh,
    out_type=jax.ShapeDtypeStruct((HIST, EMBED_DIM, BATCH), jnp.float32),
    scratch_types=[
        pltpu.VMEM((2, _STRIPE), jnp.int32),
        pltpu.VMEM((2, _STRIPE, EMBED_DIM), jnp.float32),
        pltpu.VMEM((2, EMBED_DIM, _STRIPE), jnp.float32),
        pltpu.SemaphoreType.DMA((2,)),
        pltpu.SemaphoreType.DMA((2,)),
    ],
    compiler_params=pltpu.CompilerParams(use_tc_tiling_on_sc=False,
                                         needs_layout_passes=False),
)
def _gather_kernel(idx_hbm, table_hbm, out_hbm, idx_v, rows_v, trows_v,
                   gsems, osems):
    wid = lax.axis_index("s") * 2 + lax.axis_index("c")
    col0 = wid * _STRIPE
    iota16 = lax.iota(jnp.int32, 16)

    def start_gather(h, b):
        pltpu.sync_copy(idx_hbm.at[pl.ds(h * BATCH + col0, _STRIPE)],
                        idx_v.at[b])
        pltpu.make_async_copy(table_hbm.at[idx_v.at[b]], rows_v.at[b],
                              gsems.at[b]).start()

    def out_copy(h, b):
        return pltpu.make_async_copy(
            trows_v.at[b],
            out_hbm.at[h, :, pl.ds(col0, _STRIPE)],
            osems.at[b])

    def transpose_block(b):
        @plsc.parallel_loop(0, _STRIPE // 16, 1, unroll=4)
        def jbody(j):
            rid = iota16 + j * 16
            for e in range(EMBED_DIM):
                col = jnp.full((16,), e, jnp.int32)
                v = plsc.load_gather(rows_v.at[b], [rid, col])
                trows_v.at[b][e, pl.ds(j * 16, 16)] = v

    start_gather(0, 0)

    def slot(h, b):
        @pl.when(h + 1 < HIST)
        def _():
            start_gather(h + 1, 1 - b)
        pltpu.make_async_copy(table_hbm.at[idx_v.at[b]], rows_v.at[b],
                              gsems.at[b]).wait()

        @pl.when(h >= 2)
        def _():
            out_copy(h - 2, b).wait()
        transpose_block(b)
        out_copy(h, b).start()

    def gbody(g, carry):
        slot(2 * g, 0)
        slot(2 * g + 1, 1)
        return carry

    lax.fori_loop(0, HIST // 2, gbody, 0)
    out_copy(HIST - 2, 0).wait()
    out_copy(HIST - 1, 1).wait()


def kernel(x, vectors):
    idx = x.T.reshape(-1).astype(jnp.int32)
    out = _gather_kernel(idx, vectors)
    return out.transpose(2, 0, 1)
